# Initial kernel scaffold; baseline (speedup 1.0000x reference)
import jax
import jax.numpy as jnp
from jax.experimental import pallas as pl


def kernel(ini_pred_poly, pred_polys_, gt_polys, keyPointsMask):
    raise NotImplementedError("write your pallas kernel here")


# speedup vs baseline: 2.0625x; 2.0625x over previous
import jax
import jax.numpy as jnp
from jax.experimental import pallas as pl

_B, _NP, _NG, _T = 256, 128, 128, 10
_BB = 8


def _dm_kernel(gt_ref, ini_ref, pred_ref, mask_ref, out_ref):
    gt = gt_ref[...]
    gx = gt[:, :, 0:1]
    gy = gt[:, :, 1:2]
    gxr = jnp.concatenate([gx[:, _NG - 1:_NG, :], gx[:, :_NG - 1, :]], axis=1)
    gyr = jnp.concatenate([gy[:, _NG - 1:_NG, :], gy[:, :_NG - 1, :]], axis=1)
    ix = ini_ref[:, 0:1, :]
    iy = ini_ref[:, 1:2, :]
    pxp = pred_ref[:, 0:1, :]
    pyp = pred_ref[:, 1:2, :]

    ex = gxr - ix
    ey = gyr - iy
    ux = gx - gxr
    uy = gy - gyr
    c0 = ex * ex + ey * ey
    c1 = 2.0 * (ex * ux + ey * uy)
    c2 = ux * ux + uy * uy

    m = c0
    for t in range(1, _T):
        s = t / _T
        m = jnp.minimum(m, c0 + s * (c1 + s * c2))
    mmin = jnp.min(m, axis=1, keepdims=True)
    giota = jax.lax.broadcasted_iota(jnp.int32, (_BB, _NG, _NP), 1)
    gstar = jnp.min(jnp.where(m == mmin, giota, _NG), axis=1, keepdims=True)
    oh = (giota == gstar).astype(jnp.float32)
    gxs = jnp.sum(oh * gx, axis=1, keepdims=True)
    gys = jnp.sum(oh * gy, axis=1, keepdims=True)
    gxrs = jnp.sum(oh * gxr, axis=1, keepdims=True)
    gyrs = jnp.sum(oh * gyr, axis=1, keepdims=True)
    exs = gxrs - ix
    eys = gyrs - iy
    uxs = gxs - gxrs
    uys = gys - gyrs
    c0s = exs * exs + eys * eys
    c1s = 2.0 * (exs * uxs + eys * uys)
    c2s = uxs * uxs + uys * uys
    bm = c0s
    sbest = jnp.zeros_like(bm)
    for t in range(1, _T):
        s = t / _T
        d = c0s + s * (c1s + s * c2s)
        upd = d < bm
        bm = jnp.where(upd, d, bm)
        sbest = jnp.where(upd, s, sbest)
    nx = gxs * sbest + gxrs * (1.0 - sbest)
    ny = gys * sbest + gyrs * (1.0 - sbest)
    t1 = jnp.sum(jnp.abs(pxp - nx) + jnp.abs(pyp - ny))

    d2 = c0 + (c1 + c2)
    m2 = jnp.min(d2, axis=2, keepdims=True)
    piota = jax.lax.broadcasted_iota(jnp.int32, (_BB, _NG, _NP), 2)
    pstar = jnp.min(jnp.where(d2 == m2, piota, _NP), axis=2, keepdims=True)
    oh2 = (piota == pstar).astype(jnp.float32)
    nx2 = jnp.sum(oh2 * pxp, axis=2, keepdims=True)
    ny2 = jnp.sum(oh2 * pyp, axis=2, keepdims=True)
    msk = mask_ref[...]
    t2 = jnp.sum((jnp.abs(nx2 - gx) + jnp.abs(ny2 - gy)) * msk)
    t3 = jnp.sum(msk)

    lane = jax.lax.broadcasted_iota(jnp.int32, (1, 128), 1)
    vec = (jnp.where(lane == 0, t1, 0.0)
           + jnp.where(lane == 1, t2, 0.0)
           + jnp.where(lane == 2, t3, 0.0))

    @pl.when(pl.program_id(0) == 0)
    def _():
        out_ref[...] = jnp.zeros_like(out_ref)

    out_ref[...] += vec


@jax.jit
def kernel(ini_pred_poly, pred_polys_, gt_polys, keyPointsMask):
    ini_t = jnp.transpose(ini_pred_poly, (0, 2, 1))
    pred_t = jnp.transpose(pred_polys_, (0, 2, 1))
    mask3 = keyPointsMask[:, :, None]
    sums = pl.pallas_call(
        _dm_kernel,
        grid=(_B // _BB,),
        in_specs=[
            pl.BlockSpec((_BB, _NG, 2), lambda i: (i, 0, 0)),
            pl.BlockSpec((_BB, 2, _NP), lambda i: (i, 0, 0)),
            pl.BlockSpec((_BB, 2, _NP), lambda i: (i, 0, 0)),
            pl.BlockSpec((_BB, _NG, 1), lambda i: (i, 0, 0)),
        ],
        out_specs=pl.BlockSpec((1, 128), lambda i: (0, 0)),
        out_shape=jax.ShapeDtypeStruct((1, 128), jnp.float32),
    )(gt_polys, ini_t, pred_t, mask3)
    t1 = sums[0, 0]
    t2 = sums[0, 1]
    t3 = sums[0, 2]
    loss1 = t1 / (_B * _NP * 2)
    loss = t2 / (2.0 * t3 + 1.0) + loss1
    return loss / 2.0


# speedup vs baseline: 2.3912x; 1.1593x over previous
import jax
import jax.numpy as jnp
from jax.experimental import pallas as pl

_B, _NP, _NG, _T = 256, 128, 128, 10
_BB = 8


def _dm_kernel(gt_ref, ini_ref, pred_ref, mask_ref, out_ref):
    gt = gt_ref[...]
    gx = gt[:, :, 0:1]
    gy = gt[:, :, 1:2]
    gxr = jnp.concatenate([gx[:, _NG - 1:_NG, :], gx[:, :_NG - 1, :]], axis=1)
    gyr = jnp.concatenate([gy[:, _NG - 1:_NG, :], gy[:, :_NG - 1, :]], axis=1)
    ix = ini_ref[:, 0:1, :]
    iy = ini_ref[:, 1:2, :]
    pxp = pred_ref[:, 0:1, :]
    pyp = pred_ref[:, 1:2, :]

    ex = gxr - ix
    ey = gyr - iy
    ux = gx - gxr
    uy = gy - gyr
    c0 = ex * ex + ey * ey
    c1 = 2.0 * (ex * ux + ey * uy)
    c2 = ux * ux + uy * uy

    rc2 = jnp.minimum(0.5 / c2, 1e20)
    sc = jnp.clip(c1 * (-10.0 * rc2) + 0.5, 0.0, 9.0)
    k = sc.astype(jnp.int32).astype(jnp.float32) * 0.1
    m = c0 + k * (c1 + k * c2)
    giota = jax.lax.broadcasted_iota(jnp.int32, (_BB, _NG, _NP), 1)
    mb = jax.lax.bitcast_convert_type(m, jnp.int32)
    pk = (mb & ~127) | giota
    pkmin = jnp.min(pk, axis=1, keepdims=True)
    gstar = pkmin & 127
    oh = (giota == gstar).astype(jnp.float32)
    nx = (jnp.sum(oh * gxr, axis=1, keepdims=True)
          + jnp.sum(oh * (k * ux), axis=1, keepdims=True))
    ny = (jnp.sum(oh * gyr, axis=1, keepdims=True)
          + jnp.sum(oh * (k * uy), axis=1, keepdims=True))
    t1 = jnp.sum(jnp.abs(pxp - nx) + jnp.abs(pyp - ny))

    d2 = c0 + (c1 + c2)
    piota = jax.lax.broadcasted_iota(jnp.int32, (_BB, _NG, _NP), 2)
    d2b = jax.lax.bitcast_convert_type(d2, jnp.int32)
    pk2 = (d2b & ~127) | piota
    pk2min = jnp.min(pk2, axis=2, keepdims=True)
    oh2 = (piota == (pk2min & 127)).astype(jnp.float32)
    nx2 = jnp.sum(oh2 * pxp, axis=2, keepdims=True)
    ny2 = jnp.sum(oh2 * pyp, axis=2, keepdims=True)
    msk = mask_ref[...]
    t2 = jnp.sum((jnp.abs(nx2 - gx) + jnp.abs(ny2 - gy)) * msk)
    t3 = jnp.sum(msk)

    lane = jax.lax.broadcasted_iota(jnp.int32, (1, 128), 1)
    vec = (jnp.where(lane == 0, t1, 0.0)
           + jnp.where(lane == 1, t2, 0.0)
           + jnp.where(lane == 2, t3, 0.0))

    @pl.when(pl.program_id(0) == 0)
    def _():
        out_ref[...] = jnp.zeros_like(out_ref)

    out_ref[...] += vec


@jax.jit
def kernel(ini_pred_poly, pred_polys_, gt_polys, keyPointsMask):
    ini_t = jnp.transpose(ini_pred_poly, (0, 2, 1))
    pred_t = jnp.transpose(pred_polys_, (0, 2, 1))
    mask3 = keyPointsMask[:, :, None]
    sums = pl.pallas_call(
        _dm_kernel,
        grid=(_B // _BB,),
        in_specs=[
            pl.BlockSpec((_BB, _NG, 2), lambda i: (i, 0, 0)),
            pl.BlockSpec((_BB, 2, _NP), lambda i: (i, 0, 0)),
            pl.BlockSpec((_BB, 2, _NP), lambda i: (i, 0, 0)),
            pl.BlockSpec((_BB, _NG, 1), lambda i: (i, 0, 0)),
        ],
        out_specs=pl.BlockSpec((1, 128), lambda i: (0, 0)),
        out_shape=jax.ShapeDtypeStruct((1, 128), jnp.float32),
    )(gt_polys, ini_t, pred_t, mask3)
    t1 = sums[0, 0]
    t2 = sums[0, 1]
    t3 = sums[0, 2]
    loss1 = t1 / (_B * _NP * 2)
    loss = t2 / (2.0 * t3 + 1.0) + loss1
    return loss / 2.0
